# baseline (speedup 1.0000x reference)
import jax
import jax.numpy as jnp
from jax import lax
from jax.experimental import pallas as pl
from jax.experimental.pallas import tpu as pltpu
from jax.experimental.pallas import tpu_sc as plsc

NUM_CORES = 2
NUM_SUBCORES = 16
LANES = 16
NW = NUM_CORES * NUM_SUBCORES
B = 16384
E = 64
N = 100000
EPW = E // NW
ICHUNK = 4096
NICHUNK = B // ICHUNK
BPW = B // NW
BCHUNK = 128
NBCHUNK = BPW // BCHUNK


def _sc_body(uidx_hbm, midx_hbm, uembT_hbm, membT_hbm, ubias_hbm, mbias_hbm,
             partials_hbm, bsum_hbm,
             row_v, ue_v, idx_v, idxb_v, ub_v, mb_v, bsum_v, pacc_v,
             bias_sem):
    wid = lax.axis_index("s") * NUM_CORES + lax.axis_index("c")
    base = wid * BPW
    zeros16 = jnp.zeros((LANES,), jnp.int32)

    pltpu.sync_copy(uidx_hbm.at[pl.ds(base, BPW)], idxb_v)
    bias_copies = []
    for j in range(NBCHUNK):
        sl = pl.ds(j * BCHUNK, BCHUNK)
        bias_copies.append(pltpu.async_copy(
            ubias_hbm.at[idxb_v.at[sl]], ub_v.at[sl], bias_sem))
    for cp in bias_copies:
        cp.wait()
    pltpu.sync_copy(midx_hbm.at[pl.ds(base, BPW)], idxb_v)
    bias_copies = []
    for j in range(NBCHUNK):
        sl = pl.ds(j * BCHUNK, BCHUNK)
        bias_copies.append(pltpu.async_copy(
            mbias_hbm.at[idxb_v.at[sl]], mb_v.at[sl], bias_sem))
    for cp in bias_copies:
        cp.wait()
    for k in range(BPW // LANES):
        sl = pl.ds(k * LANES, LANES)
        bsum_v[sl] = ub_v[sl] + mb_v[sl]
    pltpu.sync_copy(bsum_v, bsum_hbm.at[pl.ds(base, BPW)])

    acc = jnp.zeros((LANES,), jnp.float32)
    for ei in range(EPW):
        e = wid * EPW + ei

        pltpu.sync_copy(uembT_hbm.at[pl.ds(e, 1)], row_v)
        for t in range(NICHUNK):
            pltpu.sync_copy(uidx_hbm.at[pl.ds(t * ICHUNK, ICHUNK)], idx_v)

            def u_body(v, carry, t=t):
                idx16 = idx_v[pl.ds(v * LANES, LANES)]
                val = plsc.load_gather(row_v, [zeros16, idx16])
                ue_v[pl.ds(t * ICHUNK + v * LANES, LANES)] = val
                return carry

            lax.fori_loop(0, ICHUNK // LANES, u_body, 0)

        pltpu.sync_copy(membT_hbm.at[pl.ds(e, 1)], row_v)
        for t in range(NICHUNK):
            pltpu.sync_copy(midx_hbm.at[pl.ds(t * ICHUNK, ICHUNK)], idx_v)

            def m_body(v, a, t=t):
                idx16 = idx_v[pl.ds(v * LANES, LANES)]
                mval = plsc.load_gather(row_v, [zeros16, idx16])
                uval = ue_v[pl.ds(t * ICHUNK + v * LANES, LANES)]
                return a + mval * uval

            acc = lax.fori_loop(0, ICHUNK // LANES, m_body, acc)

    pacc_v[...] = acc
    pltpu.sync_copy(pacc_v, partials_hbm.at[wid])


def _sc_call(uidx, midx, uembT, membT, ubias, mbias):
    mesh = plsc.VectorSubcoreMesh(core_axis_name="c", subcore_axis_name="s",
                                  num_cores=NUM_CORES, num_subcores=NUM_SUBCORES)
    return pl.kernel(
        _sc_body,
        out_type=(
            jax.ShapeDtypeStruct((NW, LANES), jnp.float32),
            jax.ShapeDtypeStruct((B,), jnp.float32),
        ),
        mesh=mesh,
        compiler_params=pltpu.CompilerParams(use_tc_tiling_on_sc=False,
                                             needs_layout_passes=False),
        scratch_types=[
            pltpu.VMEM((1, N), jnp.float32),
            pltpu.VMEM((B,), jnp.float32),
            pltpu.VMEM((ICHUNK,), jnp.int32),
            pltpu.VMEM((BPW,), jnp.int32),
            pltpu.VMEM((BPW,), jnp.float32),
            pltpu.VMEM((BPW,), jnp.float32),
            pltpu.VMEM((BPW,), jnp.float32),
            pltpu.VMEM((LANES,), jnp.float32),
            pltpu.SemaphoreType.DMA,
        ],
    )(uidx, midx, uembT, membT, ubias, mbias)


def _tc_body(partials_ref, bsum_ref, out_ref):
    s = jnp.sum(partials_ref[...])
    out_ref[...] = jax.nn.sigmoid(bsum_ref[...] + s)


def _tc_call(partials, bsum2d):
    return pl.pallas_call(
        _tc_body,
        out_shape=jax.ShapeDtypeStruct(bsum2d.shape, jnp.float32),
    )(partials, bsum2d)


def kernel(inputs, user_emb, user_bias_tab, movie_emb, movie_bias_tab):
    uidx = inputs[:, 0]
    midx = inputs[:, 1]
    uembT = user_emb.T
    membT = movie_emb.T
    ubias = user_bias_tab.reshape(-1)
    mbias = movie_bias_tab.reshape(-1)
    partials, bsum = _sc_call(uidx, midx, uembT, membT, ubias, mbias)
    y = _tc_call(partials, bsum.reshape(128, 128))
    return y.reshape(B, 1)


# speedup vs baseline: 1.5298x; 1.5298x over previous
import jax
import jax.numpy as jnp
from jax import lax
from jax.experimental import pallas as pl
from jax.experimental.pallas import tpu as pltpu
from jax.experimental.pallas import tpu_sc as plsc

NUM_CORES = 2
NUM_SUBCORES = 16
LANES = 16
NW = NUM_CORES * NUM_SUBCORES
B = 16384
E = 64
N = 100000
PACKC = 2048
BPW = B // NW
HALF = BPW // 2
CHUNK = 128
NCHUNK = BPW // CHUNK


def _pack_body(ut_ref, mt_ref, out_ref):
    out_ref[...] = jnp.concatenate(
        [ut_ref[...].T, mt_ref[...].T], axis=1)


def _pack_call(uembT, membT):
    grid = (N + PACKC - 1) // PACKC
    return pl.pallas_call(
        _pack_body,
        grid=(grid,),
        in_specs=[
            pl.BlockSpec((E, PACKC), lambda i: (0, i)),
            pl.BlockSpec((E, PACKC), lambda i: (0, i)),
        ],
        out_specs=pl.BlockSpec((PACKC, 2 * E), lambda i: (i, 0)),
        out_shape=jax.ShapeDtypeStruct((N, 2 * E), jnp.float32),
    )(uembT, membT)


def _sc_body(uidx_hbm, midx_hbm, packed_hbm, ubias_hbm, mbias_hbm,
             partials_hbm, bsum_hbm,
             uidx_v, midx_v, urows_v, mrows_v, ub_v, mb_v, bsum_v, pacc_v,
             emb_sem, bias_sem):
    wid = lax.axis_index("s") * NUM_CORES + lax.axis_index("c")
    base = wid * BPW

    pltpu.sync_copy(uidx_hbm.at[pl.ds(wid * NCHUNK, NCHUNK)], uidx_v)
    pltpu.sync_copy(midx_hbm.at[pl.ds(wid * NCHUNK, NCHUNK)], midx_v)

    bias_copies = []
    for j in range(NCHUNK):
        sl = pl.ds(j * CHUNK, CHUNK)
        bias_copies.append(pltpu.async_copy(
            ubias_hbm.at[uidx_v.at[j]], ub_v.at[sl], bias_sem))
        bias_copies.append(pltpu.async_copy(
            mbias_hbm.at[midx_v.at[j]], mb_v.at[sl], bias_sem))

    zero = jnp.zeros((LANES,), jnp.float32)
    accs = (zero, zero, zero, zero)
    for h in range(BPW // HALF):
        copies = []
        for j in range(HALF // CHUNK):
            c = h * (HALF // CHUNK) + j
            sl = pl.ds(j * CHUNK, CHUNK)
            copies.append(pltpu.async_copy(
                packed_hbm.at[uidx_v.at[c]], urows_v.at[sl], emb_sem))
            copies.append(pltpu.async_copy(
                packed_hbm.at[midx_v.at[c]], mrows_v.at[sl], emb_sem))
        for cp in copies:
            cp.wait()

        def row_body(i, acc):
            out = []
            for j in range(E // LANES):
                usl = pl.ds(j * LANES, LANES)
                msl = pl.ds(E + j * LANES, LANES)
                out.append(acc[j] + urows_v[i, usl] * mrows_v[i, msl])
            return tuple(out)

        accs = lax.fori_loop(0, HALF, row_body, accs)

    pacc_v[...] = (accs[0] + accs[1]) + (accs[2] + accs[3])
    pltpu.sync_copy(pacc_v, partials_hbm.at[wid])

    for cp in bias_copies:
        cp.wait()
    for k in range(BPW // LANES):
        sl = pl.ds(k * LANES, LANES)
        bsum_v[sl] = ub_v[sl] + mb_v[sl]
    pltpu.sync_copy(bsum_v, bsum_hbm.at[pl.ds(base, BPW)])


def _sc_call(uidx, midx, packed, ubias, mbias):
    mesh = plsc.VectorSubcoreMesh(core_axis_name="c", subcore_axis_name="s",
                                  num_cores=NUM_CORES, num_subcores=NUM_SUBCORES)
    return pl.kernel(
        _sc_body,
        out_type=(
            jax.ShapeDtypeStruct((NW, LANES), jnp.float32),
            jax.ShapeDtypeStruct((B,), jnp.float32),
        ),
        mesh=mesh,
        compiler_params=pltpu.CompilerParams(use_tc_tiling_on_sc=False),
        scratch_types=[
            pltpu.VMEM((NCHUNK, CHUNK), jnp.int32),
            pltpu.VMEM((NCHUNK, CHUNK), jnp.int32),
            pltpu.VMEM((HALF, 2 * E), jnp.float32),
            pltpu.VMEM((HALF, 2 * E), jnp.float32),
            pltpu.VMEM((BPW,), jnp.float32),
            pltpu.VMEM((BPW,), jnp.float32),
            pltpu.VMEM((BPW,), jnp.float32),
            pltpu.VMEM((LANES,), jnp.float32),
            pltpu.SemaphoreType.DMA,
            pltpu.SemaphoreType.DMA,
        ],
    )(uidx, midx, packed, ubias, mbias)


def _tc_body(partials_ref, bsum_ref, out_ref):
    s = jnp.sum(partials_ref[...])
    out_ref[...] = jax.nn.sigmoid(bsum_ref[...] + s)


def _tc_call(partials, bsum2d):
    return pl.pallas_call(
        _tc_body,
        out_shape=jax.ShapeDtypeStruct(bsum2d.shape, jnp.float32),
    )(partials, bsum2d)


def kernel(inputs, user_emb, user_bias_tab, movie_emb, movie_bias_tab):
    uidx = inputs[:, 0].reshape(NW * NCHUNK, CHUNK)
    midx = inputs[:, 1].reshape(NW * NCHUNK, CHUNK)
    packed = _pack_call(user_emb.T, movie_emb.T)
    ubias = user_bias_tab.reshape(-1)
    mbias = movie_bias_tab.reshape(-1)
    partials, bsum = _sc_call(uidx, midx, packed, ubias, mbias)
    y = _tc_call(partials, bsum.reshape(128, 128))
    return y.reshape(B, 1)


# speedup vs baseline: 1.7133x; 1.1200x over previous
import jax
import jax.numpy as jnp
from jax import lax
from jax.experimental import pallas as pl
from jax.experimental.pallas import tpu as pltpu
from jax.experimental.pallas import tpu_sc as plsc

NUM_CORES = 2
NUM_SUBCORES = 16
LANES = 16
NW = NUM_CORES * NUM_SUBCORES
B = 16384
E = 64
N = 100000
PACKC = 2048
BPW = B // NW
HALF = BPW // 2
CHUNK = 128
NCHUNK = BPW // CHUNK


def _pack_body(ut_ref, mt_ref, out_ref):
    x = jnp.concatenate([ut_ref[...], mt_ref[...]], axis=0)
    ii = lax.broadcasted_iota(jnp.int32, (2 * E, 2 * E), 0)
    jj = lax.broadcasted_iota(jnp.int32, (2 * E, 2 * E), 1)
    eye = (ii == jj).astype(jnp.float32)
    out_ref[...] = lax.dot_general(
        x, eye, dimension_numbers=(((0,), (0,)), ((), ())),
        preferred_element_type=jnp.float32)


def _pack_call(uembT, membT):
    grid = (N + PACKC - 1) // PACKC
    return pl.pallas_call(
        _pack_body,
        grid=(grid,),
        in_specs=[
            pl.BlockSpec((E, PACKC), lambda i: (0, i)),
            pl.BlockSpec((E, PACKC), lambda i: (0, i)),
        ],
        out_specs=pl.BlockSpec((PACKC, 2 * E), lambda i: (i, 0)),
        out_shape=jax.ShapeDtypeStruct((N, 2 * E), jnp.float32),
    )(uembT, membT)


def _sc_body(uidx_hbm, midx_hbm, packed_hbm, ubias_hbm, mbias_hbm,
             partials_hbm, bsum_hbm,
             uidx_v, midx_v, urows_v, mrows_v, ub_v, mb_v, bsum_v, pacc_v,
             emb_sem, bias_sem):
    wid = lax.axis_index("s") * NUM_CORES + lax.axis_index("c")
    base = wid * BPW

    pltpu.sync_copy(uidx_hbm.at[pl.ds(wid * NCHUNK, NCHUNK)], uidx_v)
    pltpu.sync_copy(midx_hbm.at[pl.ds(wid * NCHUNK, NCHUNK)], midx_v)

    bias_copies = []
    for j in range(NCHUNK):
        sl = pl.ds(j * CHUNK, CHUNK)
        bias_copies.append(pltpu.async_copy(
            ubias_hbm.at[uidx_v.at[j]], ub_v.at[sl], bias_sem))
        bias_copies.append(pltpu.async_copy(
            mbias_hbm.at[midx_v.at[j]], mb_v.at[sl], bias_sem))

    zero = jnp.zeros((LANES,), jnp.float32)
    accs = (zero, zero, zero, zero)
    for h in range(BPW // HALF):
        copies = []
        for j in range(HALF // CHUNK):
            c = h * (HALF // CHUNK) + j
            sl = pl.ds(j * CHUNK, CHUNK)
            copies.append(pltpu.async_copy(
                packed_hbm.at[uidx_v.at[c]], urows_v.at[sl], emb_sem))
            copies.append(pltpu.async_copy(
                packed_hbm.at[midx_v.at[c]], mrows_v.at[sl], emb_sem))
        for cp in copies:
            cp.wait()

        def row_body(i, acc):
            out = []
            for j in range(E // LANES):
                usl = pl.ds(j * LANES, LANES)
                msl = pl.ds(E + j * LANES, LANES)
                out.append(acc[j] + urows_v[i, usl] * mrows_v[i, msl])
            return tuple(out)

        accs = lax.fori_loop(0, HALF, row_body, accs)

    pacc_v[...] = (accs[0] + accs[1]) + (accs[2] + accs[3])
    pltpu.sync_copy(pacc_v, partials_hbm.at[wid])

    for cp in bias_copies:
        cp.wait()
    for k in range(BPW // LANES):
        sl = pl.ds(k * LANES, LANES)
        bsum_v[sl] = ub_v[sl] + mb_v[sl]
    pltpu.sync_copy(bsum_v, bsum_hbm.at[pl.ds(base, BPW)])


def _sc_call(uidx, midx, packed, ubias, mbias):
    mesh = plsc.VectorSubcoreMesh(core_axis_name="c", subcore_axis_name="s",
                                  num_cores=NUM_CORES, num_subcores=NUM_SUBCORES)
    return pl.kernel(
        _sc_body,
        out_type=(
            jax.ShapeDtypeStruct((NW, LANES), jnp.float32),
            jax.ShapeDtypeStruct((B,), jnp.float32),
        ),
        mesh=mesh,
        compiler_params=pltpu.CompilerParams(use_tc_tiling_on_sc=False),
        scratch_types=[
            pltpu.VMEM((NCHUNK, CHUNK), jnp.int32),
            pltpu.VMEM((NCHUNK, CHUNK), jnp.int32),
            pltpu.VMEM((HALF, 2 * E), jnp.float32),
            pltpu.VMEM((HALF, 2 * E), jnp.float32),
            pltpu.VMEM((BPW,), jnp.float32),
            pltpu.VMEM((BPW,), jnp.float32),
            pltpu.VMEM((BPW,), jnp.float32),
            pltpu.VMEM((LANES,), jnp.float32),
            pltpu.SemaphoreType.DMA,
            pltpu.SemaphoreType.DMA,
        ],
    )(uidx, midx, packed, ubias, mbias)


def _tc_body(partials_ref, bsum_ref, out_ref):
    s = jnp.sum(partials_ref[...])
    out_ref[...] = jax.nn.sigmoid(bsum_ref[...] + s)


def _tc_call(partials, bsum2d):
    return pl.pallas_call(
        _tc_body,
        out_shape=jax.ShapeDtypeStruct(bsum2d.shape, jnp.float32),
    )(partials, bsum2d)


def kernel(inputs, user_emb, user_bias_tab, movie_emb, movie_bias_tab):
    uidx = inputs[:, 0].reshape(NW * NCHUNK, CHUNK)
    midx = inputs[:, 1].reshape(NW * NCHUNK, CHUNK)
    packed = _pack_call(user_emb.T, movie_emb.T)
    ubias = user_bias_tab.reshape(-1)
    mbias = movie_bias_tab.reshape(-1)
    partials, bsum = _sc_call(uidx, midx, packed, ubias, mbias)
    y = _tc_call(partials, bsum.reshape(128, 128))
    return y.reshape(B, 1)


# speedup vs baseline: 2.2131x; 1.2917x over previous
import jax
import jax.numpy as jnp
from jax import lax
from jax.experimental import pallas as pl
from jax.experimental.pallas import tpu as pltpu
from jax.experimental.pallas import tpu_sc as plsc

NUM_CORES = 2
NUM_SUBCORES = 16
LANES = 16
NW = NUM_CORES * NUM_SUBCORES
B = 16384
E = 64
N = 100000
PACKC = 8192
BPW = B // NW
HALF = BPW // 2
CHUNK = 128
NCHUNK = BPW // CHUNK

_SC_MESH = dict(core_axis_name="c", subcore_axis_name="s",
                num_cores=NUM_CORES, num_subcores=NUM_SUBCORES)
_SC_PARAMS = pltpu.CompilerParams(use_tc_tiling_on_sc=False)


def _pack_body(ut_ref, mt_ref, out_ref):
    x = jnp.concatenate([ut_ref[...], mt_ref[...]], axis=0)
    ii = lax.broadcasted_iota(jnp.int32, (2 * E, 2 * E), 0)
    jj = lax.broadcasted_iota(jnp.int32, (2 * E, 2 * E), 1)
    eye = (ii == jj).astype(jnp.float32)
    out_ref[...] = lax.dot_general(
        x, eye, dimension_numbers=(((0,), (0,)), ((), ())),
        preferred_element_type=jnp.float32)


def _pack_call(uembT, membT):
    grid = (N + PACKC - 1) // PACKC
    return pl.pallas_call(
        _pack_body,
        grid=(grid,),
        in_specs=[
            pl.BlockSpec((E, PACKC), lambda i: (0, i)),
            pl.BlockSpec((E, PACKC), lambda i: (0, i)),
        ],
        out_specs=pl.BlockSpec((PACKC, 2 * E), lambda i: (i, 0)),
        out_shape=jax.ShapeDtypeStruct((N, 2 * E), jnp.float32),
    )(uembT, membT)


def _bias_body(uidx_hbm, midx_hbm, ubias_hbm, mbias_hbm, bsum_hbm,
               uidx_v, midx_v, ub_v, mb_v, bsum_v, bias_sem):
    wid = lax.axis_index("s") * NUM_CORES + lax.axis_index("c")
    base = wid * BPW
    pltpu.sync_copy(uidx_hbm.at[pl.ds(wid * NCHUNK, NCHUNK)], uidx_v)
    pltpu.sync_copy(midx_hbm.at[pl.ds(wid * NCHUNK, NCHUNK)], midx_v)
    copies = []
    for j in range(NCHUNK):
        sl = pl.ds(j * CHUNK, CHUNK)
        copies.append(pltpu.async_copy(
            ubias_hbm.at[uidx_v.at[j]], ub_v.at[sl], bias_sem))
        copies.append(pltpu.async_copy(
            mbias_hbm.at[midx_v.at[j]], mb_v.at[sl], bias_sem))
    for cp in copies:
        cp.wait()
    for k in range(BPW // LANES):
        sl = pl.ds(k * LANES, LANES)
        bsum_v[sl] = ub_v[sl] + mb_v[sl]
    pltpu.sync_copy(bsum_v, bsum_hbm.at[pl.ds(base, BPW)])


def _bias_call(uidx, midx, ubias, mbias):
    return pl.kernel(
        _bias_body,
        out_type=jax.ShapeDtypeStruct((B,), jnp.float32),
        mesh=plsc.VectorSubcoreMesh(**_SC_MESH),
        compiler_params=_SC_PARAMS,
        scratch_types=[
            pltpu.VMEM((NCHUNK, CHUNK), jnp.int32),
            pltpu.VMEM((NCHUNK, CHUNK), jnp.int32),
            pltpu.VMEM((BPW,), jnp.float32),
            pltpu.VMEM((BPW,), jnp.float32),
            pltpu.VMEM((BPW,), jnp.float32),
            pltpu.SemaphoreType.DMA,
        ],
    )(uidx, midx, ubias, mbias)


def _dot_body(uidx_hbm, midx_hbm, packed_hbm, partials_hbm,
              uidx_v, midx_v, urows_v, mrows_v, pacc_v, emb_sem):
    wid = lax.axis_index("s") * NUM_CORES + lax.axis_index("c")

    pltpu.sync_copy(uidx_hbm.at[pl.ds(wid * NCHUNK, NCHUNK)], uidx_v)
    pltpu.sync_copy(midx_hbm.at[pl.ds(wid * NCHUNK, NCHUNK)], midx_v)

    zero = jnp.zeros((LANES,), jnp.float32)
    accs = (zero, zero, zero, zero)
    for h in range(BPW // HALF):
        copies = []
        for j in range(HALF // CHUNK):
            c = h * (HALF // CHUNK) + j
            sl = pl.ds(j * CHUNK, CHUNK)
            copies.append(pltpu.async_copy(
                packed_hbm.at[uidx_v.at[c]], urows_v.at[sl], emb_sem))
            copies.append(pltpu.async_copy(
                packed_hbm.at[midx_v.at[c]], mrows_v.at[sl], emb_sem))
        for cp in copies:
            cp.wait()

        def row_body(i, acc):
            out = []
            for j in range(E // LANES):
                usl = pl.ds(j * LANES, LANES)
                msl = pl.ds(E + j * LANES, LANES)
                out.append(acc[j] + urows_v[i, usl] * mrows_v[i, msl])
            return tuple(out)

        accs = lax.fori_loop(0, HALF, row_body, accs)

    pacc_v[...] = (accs[0] + accs[1]) + (accs[2] + accs[3])
    pltpu.sync_copy(pacc_v, partials_hbm.at[wid])


def _dot_call(uidx, midx, packed):
    return pl.kernel(
        _dot_body,
        out_type=jax.ShapeDtypeStruct((NW, LANES), jnp.float32),
        mesh=plsc.VectorSubcoreMesh(**_SC_MESH),
        compiler_params=_SC_PARAMS,
        scratch_types=[
            pltpu.VMEM((NCHUNK, CHUNK), jnp.int32),
            pltpu.VMEM((NCHUNK, CHUNK), jnp.int32),
            pltpu.VMEM((HALF, 2 * E), jnp.float32),
            pltpu.VMEM((HALF, 2 * E), jnp.float32),
            pltpu.VMEM((LANES,), jnp.float32),
            pltpu.SemaphoreType.DMA,
        ],
    )(uidx, midx, packed)


def _tc_body(partials_ref, bsum_ref, out_ref):
    s = jnp.sum(partials_ref[...])
    out_ref[...] = jax.nn.sigmoid(bsum_ref[...] + s)


def _tc_call(partials, bsum2d):
    return pl.pallas_call(
        _tc_body,
        out_shape=jax.ShapeDtypeStruct(bsum2d.shape, jnp.float32),
    )(partials, bsum2d)


def kernel(inputs, user_emb, user_bias_tab, movie_emb, movie_bias_tab):
    uidx = inputs[:, 0].reshape(NW * NCHUNK, CHUNK)
    midx = inputs[:, 1].reshape(NW * NCHUNK, CHUNK)
    ubias = user_bias_tab.reshape(-1)
    mbias = movie_bias_tab.reshape(-1)
    bsum = _bias_call(uidx, midx, ubias, mbias)
    packed = _pack_call(user_emb.T, movie_emb.T)
    partials = _dot_call(uidx, midx, packed)
    y = _tc_call(partials, bsum.reshape(128, 128))
    return y.reshape(B, 1)


# speedup vs baseline: 2.2527x; 1.0179x over previous
import jax
import jax.numpy as jnp
from jax import lax
from jax.experimental import pallas as pl
from jax.experimental.pallas import tpu as pltpu
from jax.experimental.pallas import tpu_sc as plsc

NUM_CORES = 2
NUM_SUBCORES = 16
LANES = 16
NW = NUM_CORES * NUM_SUBCORES
B = 16384
E = 64
N = 100000
PACKC = 8192
BPW = B // NW
HALF = BPW // 2
CHUNK = 128
NCHUNK = BPW // CHUNK

_SC_MESH = dict(core_axis_name="c", subcore_axis_name="s",
                num_cores=NUM_CORES, num_subcores=NUM_SUBCORES)
_SC_PARAMS = pltpu.CompilerParams(use_tc_tiling_on_sc=False)


def _pack_body(ut_ref, mt_ref, out_ref):
    x = jnp.concatenate([ut_ref[...], mt_ref[...]], axis=0)
    ii = lax.broadcasted_iota(jnp.int32, (2 * E, 2 * E), 0)
    jj = lax.broadcasted_iota(jnp.int32, (2 * E, 2 * E), 1)
    eye = (ii == jj).astype(jnp.float32)
    out_ref[...] = lax.dot_general(
        x, eye, dimension_numbers=(((0,), (0,)), ((), ())),
        preferred_element_type=jnp.float32)


def _pack_call(uembT, membT):
    grid = (N + PACKC - 1) // PACKC
    return pl.pallas_call(
        _pack_body,
        grid=(grid,),
        in_specs=[
            pl.BlockSpec((E, PACKC), lambda i: (0, i)),
            pl.BlockSpec((E, PACKC), lambda i: (0, i)),
        ],
        out_specs=pl.BlockSpec((PACKC, 2 * E), lambda i: (i, 0)),
        out_shape=jax.ShapeDtypeStruct((N, 2 * E), jnp.float32),
    )(uembT, membT)


def _sc_body(uidx_hbm, midx_hbm, packed_hbm, ubias_hbm, mbias_hbm,
             partials_hbm, bsum_hbm,
             uidx_v, midx_v, urows0_v, mrows0_v, urows1_v, mrows1_v,
             ub_v, mb_v, bsum_v, pacc_v,
             sem0, sem1, bias_sem):
    wid = lax.axis_index("s") * NUM_CORES + lax.axis_index("c")
    base = wid * BPW

    pltpu.sync_copy(uidx_hbm.at[pl.ds(wid * NCHUNK, NCHUNK)], uidx_v)
    pltpu.sync_copy(midx_hbm.at[pl.ds(wid * NCHUNK, NCHUNK)], midx_v)

    bias_copies = []
    for j in range(NCHUNK):
        sl = pl.ds(j * CHUNK, CHUNK)
        bias_copies.append(pltpu.async_copy(
            ubias_hbm.at[uidx_v.at[j]], ub_v.at[sl], bias_sem))
        bias_copies.append(pltpu.async_copy(
            mbias_hbm.at[midx_v.at[j]], mb_v.at[sl], bias_sem))

    bufs = ((urows0_v, mrows0_v, sem0), (urows1_v, mrows1_v, sem1))

    def fire(q):
        urows_v, mrows_v, sem = bufs[q % 2]
        return (pltpu.async_copy(
                    packed_hbm.at[uidx_v.at[q]], urows_v, sem),
                pltpu.async_copy(
                    packed_hbm.at[midx_v.at[q]], mrows_v, sem))

    zero = jnp.zeros((LANES,), jnp.float32)
    accs = (zero, zero, zero, zero)
    inflight = [fire(0), fire(1)]
    for q in range(NCHUNK):
        urows_v, mrows_v, sem = bufs[q % 2]
        for cp in inflight[q]:
            cp.wait()
        if q + 2 < NCHUNK:
            inflight.append(fire(q + 2))

        def row_body(i, acc, urows_v=urows_v, mrows_v=mrows_v):
            out = []
            for j in range(E // LANES):
                usl = pl.ds(j * LANES, LANES)
                msl = pl.ds(E + j * LANES, LANES)
                out.append(acc[j] + urows_v[i, usl] * mrows_v[i, msl])
            return tuple(out)

        accs = lax.fori_loop(0, CHUNK, row_body, accs)

    pacc_v[...] = (accs[0] + accs[1]) + (accs[2] + accs[3])
    pltpu.sync_copy(pacc_v, partials_hbm.at[wid])

    for cp in bias_copies:
        cp.wait()
    for k in range(BPW // LANES):
        sl = pl.ds(k * LANES, LANES)
        bsum_v[sl] = ub_v[sl] + mb_v[sl]
    pltpu.sync_copy(bsum_v, bsum_hbm.at[pl.ds(base, BPW)])


def _sc_call(uidx, midx, packed, ubias, mbias):
    return pl.kernel(
        _sc_body,
        out_type=(
            jax.ShapeDtypeStruct((NW, LANES), jnp.float32),
            jax.ShapeDtypeStruct((B,), jnp.float32),
        ),
        mesh=plsc.VectorSubcoreMesh(**_SC_MESH),
        compiler_params=_SC_PARAMS,
        scratch_types=[
            pltpu.VMEM((NCHUNK, CHUNK), jnp.int32),
            pltpu.VMEM((NCHUNK, CHUNK), jnp.int32),
            pltpu.VMEM((CHUNK, 2 * E), jnp.float32),
            pltpu.VMEM((CHUNK, 2 * E), jnp.float32),
            pltpu.VMEM((CHUNK, 2 * E), jnp.float32),
            pltpu.VMEM((CHUNK, 2 * E), jnp.float32),
            pltpu.VMEM((BPW,), jnp.float32),
            pltpu.VMEM((BPW,), jnp.float32),
            pltpu.VMEM((BPW,), jnp.float32),
            pltpu.VMEM((LANES,), jnp.float32),
            pltpu.SemaphoreType.DMA,
            pltpu.SemaphoreType.DMA,
            pltpu.SemaphoreType.DMA,
        ],
    )(uidx, midx, packed, ubias, mbias)


def _tc_body(partials_ref, bsum_ref, out_ref):
    s = jnp.sum(partials_ref[...])
    out_ref[...] = jax.nn.sigmoid(bsum_ref[...] + s)


def _tc_call(partials, bsum2d):
    return pl.pallas_call(
        _tc_body,
        out_shape=jax.ShapeDtypeStruct(bsum2d.shape, jnp.float32),
    )(partials, bsum2d)


def kernel(inputs, user_emb, user_bias_tab, movie_emb, movie_bias_tab):
    uidx = inputs[:, 0].reshape(NW * NCHUNK, CHUNK)
    midx = inputs[:, 1].reshape(NW * NCHUNK, CHUNK)
    ubias = user_bias_tab.reshape(-1)
    mbias = movie_bias_tab.reshape(-1)
    packed = _pack_call(user_emb.T, movie_emb.T)
    partials, bsum = _sc_call(uidx, midx, packed, ubias, mbias)
    y = _tc_call(partials, bsum.reshape(128, 128))
    return y.reshape(B, 1)
